# baseline (speedup 1.0000x reference)
import functools

import jax
import jax.numpy as jnp
from jax import lax
from jax.experimental import pallas as pl
from jax.experimental.pallas import tpu as pltpu
from jax.experimental.pallas import tpu_sc as plsc

N = 10000
E = 320000
D = 128
NPAD = 10240
NC, NS = 2, 16
NW = NC * NS
C = 128
NCHT = 80
NCH = NCHT // 2
EPAD = NW * NCHT * C
ROWS = EPAD // C
ROWSH = ROWS // 2
RPT = NPAD // NS

_mesh = plsc.VectorSubcoreMesh(core_axis_name="c", subcore_axis_name="s")


def _sc_degrees(s0, d0, s1, d1):
  @functools.partial(
      pl.kernel,
      out_type=[jax.ShapeDtypeStruct((2 * NPAD,), jnp.float32)] * 4,
      mesh=_mesh,
      scratch_types=[
          pltpu.VMEM((4 * NCHT, C), jnp.int32),
          pltpu.VMEM((C,), jnp.float32),
          pltpu.VMEM((RPT,), jnp.float32),
          pltpu.VMEM_SHARED((NPAD,), jnp.float32),
          pltpu.VMEM_SHARED((NPAD,), jnp.float32),
          pltpu.VMEM_SHARED((NPAD,), jnp.float32),
          pltpu.VMEM_SHARED((NPAD,), jnp.float32),
      ],
  )
  def deg_kernel(s0_hbm, d0_hbm, s1_hbm, d1_hbm, o0, o1, o2, o3,
                 idx_v, ones_v, zeros_v, h0, h1, h2, h3):
    core = lax.axis_index("c")
    sub = lax.axis_index("s")
    wid = sub * NC + core
    hists = (h0, h1, h2, h3)
    outs = (o0, o1, o2, o3)

    @pl.loop(0, C // 16)
    def _(t):
      ones_v[pl.ds(t * 16, 16)] = jnp.ones((16,), jnp.float32)

    @pl.loop(0, RPT // 16)
    def _(t):
      zeros_v[pl.ds(t * 16, 16)] = jnp.zeros((16,), jnp.float32)

    for h in hists:
      pltpu.sync_copy(zeros_v, h.at[pl.ds(sub * RPT, RPT)])

    for k, src in enumerate((s0_hbm, d0_hbm, s1_hbm, d1_hbm)):
      pltpu.sync_copy(src.at[pl.ds(wid * NCHT, NCHT)],
                      idx_v.at[pl.ds(k * NCHT, NCHT)])

    plsc.subcore_barrier()

    for k, h in enumerate(hists):
      @pl.loop(0, NCHT)
      def _(j, k=k, h=h):
        pltpu.sync_copy(ones_v, h.at[idx_v.at[k * NCHT + j]], add=True)

    plsc.subcore_barrier()

    for h, o in zip(hists, outs):
      pltpu.sync_copy(h.at[pl.ds(sub * RPT, RPT)],
                      o.at[pl.ds(core * NPAD + sub * RPT, RPT)])

  return deg_kernel(s0, d0, s1, d1)


def _sc_aggregate(h, src, dst):
  @functools.partial(
      pl.kernel,
      out_type=jax.ShapeDtypeStruct((2, NPAD, D), jnp.float32),
      mesh=_mesh,
      scratch_types=[
          pltpu.VMEM((NCH, C), jnp.int32),
          pltpu.VMEM((NCH, C), jnp.int32),
          pltpu.VMEM((2 * C, D), jnp.float32),
          pltpu.VMEM_SHARED((NPAD, D), jnp.float32),
          pltpu.SemaphoreType.DMA,
      ],
  )
  def agg_kernel(h_hbm, src_hbm, dst_hbm, out_hbm,
                 sidx, didx, buf, acc, sem0):
    core = lax.axis_index("c")
    sub = lax.axis_index("s")
    wid = sub * NC + core

    pltpu.sync_copy(src_hbm.at[pl.ds(wid * NCH, NCH)], sidx)
    pltpu.sync_copy(dst_hbm.at[pl.ds(wid * NCH, NCH)], didx)

    @pl.loop(0, 2 * C)
    def _(r):
      @pl.loop(0, D // 16)
      def _(t):
        buf.at[pl.ds(r, 1), pl.ds(t * 16, 16)][...] = (
            jnp.zeros((1, 16), jnp.float32))

    @pl.loop(0, RPT // C)
    def _(b):
      pltpu.sync_copy(buf.at[pl.ds(0, C)],
                      acc.at[pl.ds(sub * RPT + b * C, C)])

    plsc.subcore_barrier()

    @pl.loop(0, NCH + 1)
    def _(i):
      @pl.when(i < NCH)
      def _():
        pltpu.async_copy(h_hbm.at[sidx.at[i]],
                         buf.at[pl.ds((i % 2) * C, C)], sem0)

      @pl.when(i >= 1)
      def _():
        j = i - 1
        sl = buf.at[pl.ds((j % 2) * C, C)]
        pltpu.make_async_copy(h_hbm.at[sidx.at[0]], sl, sem0).wait()
        pltpu.sync_copy(sl, acc.at[didx.at[j]], add=True)

    plsc.subcore_barrier()

    pltpu.sync_copy(acc.at[pl.ds(sub * RPT, RPT)],
                    out_hbm.at[core, pl.ds(sub * RPT, RPT)])

  return agg_kernel(h, src, dst)


def _norm_body(p0, p1, p2, p3, o0, o1, o2, o3):
  for p, o in ((p0, o0), (p1, o1), (p2, o2), (p3, o3)):
    deg = p[pl.ds(0, NPAD)] + p[pl.ds(NPAD, NPAD)]
    o[...] = jnp.where(deg > 0, lax.rsqrt(jnp.maximum(deg, 1.0)), 0.0)


def _norms(degp):
  return pl.pallas_call(
      _norm_body,
      out_shape=[jax.ShapeDtypeStruct((NPAD,), jnp.float32)] * 4,
  )(*degp)


BM = 1024


def _mm1_body(x_ref, w_ref, ns_ref, o_ref):
  o_ref[...] = jnp.dot(x_ref[...], w_ref[...],
                       preferred_element_type=jnp.float32) * ns_ref[...]


def _tc_mm1(x, W, ns):
  return pl.pallas_call(
      _mm1_body,
      grid=(NPAD // BM,),
      in_specs=[
          pl.BlockSpec((BM, D), lambda i: (i, 0)),
          pl.BlockSpec((D, D), lambda i: (0, 0)),
          pl.BlockSpec((BM, 1), lambda i: (i, 0)),
      ],
      out_specs=pl.BlockSpec((BM, D), lambda i: (i, 0)),
      out_shape=jax.ShapeDtypeStruct((NPAD, D), jnp.float32),
  )(x, W, ns)


def _mm2_body(pa_ref, pb_ref, nd_ref, b_ref, w_ref, ns_ref, o_ref):
  agg = pa_ref[0] + pa_ref[1] + pb_ref[0] + pb_ref[1]
  x2 = jnp.maximum(agg * nd_ref[...] + b_ref[...], 0.0)
  o_ref[...] = jnp.dot(x2, w_ref[...],
                       preferred_element_type=jnp.float32) * ns_ref[...]


def _tc_mm2(pa, pb, nd, b, W, ns):
  return pl.pallas_call(
      _mm2_body,
      grid=(NPAD // BM,),
      in_specs=[
          pl.BlockSpec((2, BM, D), lambda i: (0, i, 0)),
          pl.BlockSpec((2, BM, D), lambda i: (0, i, 0)),
          pl.BlockSpec((BM, 1), lambda i: (i, 0)),
          pl.BlockSpec((1, D), lambda i: (0, 0)),
          pl.BlockSpec((D, D), lambda i: (0, 0)),
          pl.BlockSpec((BM, 1), lambda i: (i, 0)),
      ],
      out_specs=pl.BlockSpec((BM, D), lambda i: (i, 0)),
      out_shape=jax.ShapeDtypeStruct((NPAD, D), jnp.float32),
  )(pa, pb, nd, b, W, ns)


def _fin_body(pa_ref, pb_ref, nd_ref, b_ref, o_ref):
  agg = pa_ref[0] + pa_ref[1] + pb_ref[0] + pb_ref[1]
  o_ref[...] = agg * nd_ref[...] + b_ref[...]


def _tc_fin(pa, pb, nd, b):
  return pl.pallas_call(
      _fin_body,
      grid=(NPAD // BM,),
      in_specs=[
          pl.BlockSpec((2, BM, D), lambda i: (0, i, 0)),
          pl.BlockSpec((2, BM, D), lambda i: (0, i, 0)),
          pl.BlockSpec((BM, 1), lambda i: (i, 0)),
          pl.BlockSpec((1, D), lambda i: (0, 0)),
      ],
      out_specs=pl.BlockSpec((BM, D), lambda i: (i, 0)),
      out_shape=jax.ShapeDtypeStruct((NPAD, D), jnp.float32),
  )(pa, pb, nd, b)


def kernel(features, edge_index0, edge_index1, W1, b1, W2, b2):
  x = jnp.pad(features, ((0, NPAD - N), (0, 0)))
  pad_ids = (N + jnp.arange(EPAD - E, dtype=jnp.int32) % (NPAD - N))

  def prep(idx):
    return jnp.concatenate([idx, pad_ids]).reshape(ROWS, C)

  s0 = prep(edge_index0[0])
  d0 = prep(edge_index0[1])
  s1 = prep(edge_index1[0])
  d1 = prep(edge_index1[1])

  degp = _sc_degrees(s0, d0, s1, d1)
  norms = _norms(degp)
  ns0 = norms[0].reshape(NPAD, 1)
  nd0 = norms[1].reshape(NPAD, 1)
  ns1 = norms[2].reshape(NPAD, 1)
  nd1 = norms[3].reshape(NPAD, 1)

  h1 = _tc_mm1(x, W1, ns0)
  p1a = _sc_aggregate(h1, s0[:ROWSH], d0[:ROWSH])
  p1b = _sc_aggregate(h1, s0[ROWSH:], d0[ROWSH:])
  h2 = _tc_mm2(p1a, p1b, nd0, b1.reshape(1, D), W2, ns1)
  p2a = _sc_aggregate(h2, s1[:ROWSH], d1[:ROWSH])
  p2b = _sc_aggregate(h2, s1[ROWSH:], d1[ROWSH:])
  out = _tc_fin(p2a, p2b, nd1, b2.reshape(1, D))
  return out[:N]


# speedup vs baseline: 1.0219x; 1.0219x over previous
import functools

import jax
import jax.numpy as jnp
from jax import lax
from jax.experimental import pallas as pl
from jax.experimental.pallas import tpu as pltpu
from jax.experimental.pallas import tpu_sc as plsc

N = 10000
E = 320000
D = 128
NPAD = 10240
NC, NS = 2, 16
NW = NC * NS
C = 128
NCHT = 80
NCH = NCHT // 2
EPAD = NW * NCHT * C
ROWS = EPAD // C
ROWSH = ROWS // 2
RPT = NPAD // NS

_mesh = plsc.VectorSubcoreMesh(core_axis_name="c", subcore_axis_name="s")


def _sc_degrees(s0, d0, s1, d1):
  @functools.partial(
      pl.kernel,
      out_type=[jax.ShapeDtypeStruct((2 * NPAD,), jnp.float32)] * 4,
      mesh=_mesh,
      scratch_types=[
          pltpu.VMEM((4 * NCHT, C), jnp.int32),
          pltpu.VMEM((C,), jnp.float32),
          pltpu.VMEM((RPT,), jnp.float32),
          pltpu.VMEM_SHARED((NPAD,), jnp.float32),
          pltpu.VMEM_SHARED((NPAD,), jnp.float32),
          pltpu.VMEM_SHARED((NPAD,), jnp.float32),
          pltpu.VMEM_SHARED((NPAD,), jnp.float32),
      ],
  )
  def deg_kernel(s0_hbm, d0_hbm, s1_hbm, d1_hbm, o0, o1, o2, o3,
                 idx_v, ones_v, zeros_v, h0, h1, h2, h3):
    core = lax.axis_index("c")
    sub = lax.axis_index("s")
    wid = sub * NC + core
    hists = (h0, h1, h2, h3)
    outs = (o0, o1, o2, o3)

    @pl.loop(0, C // 16)
    def _(t):
      ones_v[pl.ds(t * 16, 16)] = jnp.ones((16,), jnp.float32)

    @pl.loop(0, RPT // 16)
    def _(t):
      zeros_v[pl.ds(t * 16, 16)] = jnp.zeros((16,), jnp.float32)

    for h in hists:
      pltpu.sync_copy(zeros_v, h.at[pl.ds(sub * RPT, RPT)])

    for k, src in enumerate((s0_hbm, d0_hbm, s1_hbm, d1_hbm)):
      pltpu.sync_copy(src.at[pl.ds(wid * NCHT, NCHT)],
                      idx_v.at[pl.ds(k * NCHT, NCHT)])

    plsc.subcore_barrier()

    for k, h in enumerate(hists):
      @pl.loop(0, NCHT)
      def _(j, k=k, h=h):
        pltpu.sync_copy(ones_v, h.at[idx_v.at[k * NCHT + j]], add=True)

    plsc.subcore_barrier()

    for h, o in zip(hists, outs):
      pltpu.sync_copy(h.at[pl.ds(sub * RPT, RPT)],
                      o.at[pl.ds(core * NPAD + sub * RPT, RPT)])

  return deg_kernel(s0, d0, s1, d1)


def _sc_aggregate(h, src, dst):
  @functools.partial(
      pl.kernel,
      out_type=jax.ShapeDtypeStruct((2, NPAD, D), jnp.float32),
      mesh=_mesh,
      scratch_types=[
          pltpu.VMEM((NCH, C), jnp.int32),
          pltpu.VMEM((NCH, C), jnp.int32),
          pltpu.VMEM((2 * C, D), jnp.float32),
          pltpu.VMEM_SHARED((NPAD, D), jnp.float32),
          pltpu.SemaphoreType.DMA,
      ],
  )
  def agg_kernel(h_hbm, src_hbm, dst_hbm, out_hbm,
                 sidx, didx, buf, acc, sem0):
    core = lax.axis_index("c")
    sub = lax.axis_index("s")
    wid = sub * NC + core

    pltpu.sync_copy(src_hbm.at[pl.ds(wid * NCH, NCH)], sidx)
    pltpu.sync_copy(dst_hbm.at[pl.ds(wid * NCH, NCH)], didx)

    @pl.loop(0, 2 * C)
    def _(r):
      @pl.loop(0, D // 16)
      def _(t):
        buf.at[pl.ds(r, 1), pl.ds(t * 16, 16)][...] = (
            jnp.zeros((1, 16), jnp.float32))

    @pl.loop(0, RPT // C)
    def _(b):
      pltpu.sync_copy(buf.at[pl.ds(0, C)],
                      acc.at[pl.ds(sub * RPT + b * C, C)])

    plsc.subcore_barrier()

    @pl.loop(0, NCH + 1)
    def _(i):
      @pl.when(i < NCH)
      def _():
        pltpu.async_copy(h_hbm.at[sidx.at[i]],
                         buf.at[pl.ds((i % 2) * C, C)], sem0)

      @pl.when(i >= 1)
      def _():
        j = i - 1
        sl = buf.at[pl.ds((j % 2) * C, C)]
        pltpu.make_async_copy(h_hbm.at[sidx.at[0]], sl, sem0).wait()
        pltpu.sync_copy(sl, acc.at[didx.at[j]], add=True)

    plsc.subcore_barrier()

    pltpu.sync_copy(acc.at[pl.ds(sub * RPT, RPT)],
                    out_hbm.at[core, pl.ds(sub * RPT, RPT)])

  return agg_kernel(h, src, dst)


BM = 1024


def _norm(p_ref):
  deg = p_ref[0] + p_ref[1]
  n = jnp.where(deg > 0, lax.rsqrt(jnp.maximum(deg, 1.0)), 0.0)
  return n.reshape(-1, 1)


_DEG_SPEC = pl.BlockSpec((2, BM), lambda i: (0, i))


def _mm1_body(x_ref, w_ref, ds_ref, o_ref):
  o_ref[...] = jnp.dot(x_ref[...], w_ref[...],
                       preferred_element_type=jnp.float32) * _norm(ds_ref)


def _tc_mm1(x, W, degs):
  return pl.pallas_call(
      _mm1_body,
      grid=(NPAD // BM,),
      in_specs=[
          pl.BlockSpec((BM, D), lambda i: (i, 0)),
          pl.BlockSpec((D, D), lambda i: (0, 0)),
          _DEG_SPEC,
      ],
      out_specs=pl.BlockSpec((BM, D), lambda i: (i, 0)),
      out_shape=jax.ShapeDtypeStruct((NPAD, D), jnp.float32),
  )(x, W, degs)


def _mm2_body(pa_ref, pb_ref, dd_ref, b_ref, w_ref, ds_ref, o_ref):
  agg = pa_ref[0] + pa_ref[1] + pb_ref[0] + pb_ref[1]
  x2 = jnp.maximum(agg * _norm(dd_ref) + b_ref[...], 0.0)
  o_ref[...] = jnp.dot(x2, w_ref[...],
                       preferred_element_type=jnp.float32) * _norm(ds_ref)


def _tc_mm2(pa, pb, degd, b, W, degs):
  return pl.pallas_call(
      _mm2_body,
      grid=(NPAD // BM,),
      in_specs=[
          pl.BlockSpec((2, BM, D), lambda i: (0, i, 0)),
          pl.BlockSpec((2, BM, D), lambda i: (0, i, 0)),
          _DEG_SPEC,
          pl.BlockSpec((1, D), lambda i: (0, 0)),
          pl.BlockSpec((D, D), lambda i: (0, 0)),
          _DEG_SPEC,
      ],
      out_specs=pl.BlockSpec((BM, D), lambda i: (i, 0)),
      out_shape=jax.ShapeDtypeStruct((NPAD, D), jnp.float32),
  )(pa, pb, degd, b, W, degs)


def _fin_body(pa_ref, pb_ref, dd_ref, b_ref, o_ref):
  agg = pa_ref[0] + pa_ref[1] + pb_ref[0] + pb_ref[1]
  o_ref[...] = agg * _norm(dd_ref) + b_ref[...]


def _tc_fin(pa, pb, degd, b):
  return pl.pallas_call(
      _fin_body,
      grid=(NPAD // BM,),
      in_specs=[
          pl.BlockSpec((2, BM, D), lambda i: (0, i, 0)),
          pl.BlockSpec((2, BM, D), lambda i: (0, i, 0)),
          _DEG_SPEC,
          pl.BlockSpec((1, D), lambda i: (0, 0)),
      ],
      out_specs=pl.BlockSpec((BM, D), lambda i: (i, 0)),
      out_shape=jax.ShapeDtypeStruct((NPAD, D), jnp.float32),
  )(pa, pb, degd, b)


def kernel(features, edge_index0, edge_index1, W1, b1, W2, b2):
  x = jnp.pad(features, ((0, NPAD - N), (0, 0)))
  pad_ids = (N + jnp.arange(EPAD - E, dtype=jnp.int32) % (NPAD - N))

  def prep(idx):
    return jnp.concatenate([idx, pad_ids]).reshape(ROWS, C)

  s0 = prep(edge_index0[0])
  d0 = prep(edge_index0[1])
  s1 = prep(edge_index1[0])
  d1 = prep(edge_index1[1])

  degp = _sc_degrees(s0, d0, s1, d1)
  dg = [p.reshape(2, NPAD) for p in degp]

  h1 = _tc_mm1(x, W1, dg[0])
  p1a = _sc_aggregate(h1, s0[:ROWSH], d0[:ROWSH])
  p1b = _sc_aggregate(h1, s0[ROWSH:], d0[ROWSH:])
  h2 = _tc_mm2(p1a, p1b, dg[1], b1.reshape(1, D), W2, dg[2])
  p2a = _sc_aggregate(h2, s1[:ROWSH], d1[:ROWSH])
  p2b = _sc_aggregate(h2, s1[ROWSH:], d1[ROWSH:])
  out = _tc_fin(p2a, p2b, dg[3], b2.reshape(1, D))
  return out[:N]


# speedup vs baseline: 1.0307x; 1.0086x over previous
import functools

import jax
import jax.numpy as jnp
from jax import lax
from jax.experimental import pallas as pl
from jax.experimental.pallas import tpu as pltpu
from jax.experimental.pallas import tpu_sc as plsc

N = 10000
E = 320000
D = 128
NPAD = 10240
NC, NS = 2, 16
NW = NC * NS
C = 128
NCHT = 80
NCH = NCHT // 2
EPAD = NW * NCHT * C
ROWS = EPAD // C
ROWSH = ROWS // 2
RPT = NPAD // NS

_mesh = plsc.VectorSubcoreMesh(core_axis_name="c", subcore_axis_name="s")


def _sc_degrees(s0, d0, s1, d1):
  @functools.partial(
      pl.kernel,
      out_type=[jax.ShapeDtypeStruct((2 * NPAD,), jnp.float32)] * 4,
      mesh=_mesh,
      scratch_types=[
          pltpu.VMEM((4 * NCHT, C), jnp.int32),
          pltpu.VMEM((C,), jnp.float32),
          pltpu.VMEM((RPT,), jnp.float32),
          pltpu.VMEM_SHARED((NPAD,), jnp.float32),
          pltpu.VMEM_SHARED((NPAD,), jnp.float32),
          pltpu.VMEM_SHARED((NPAD,), jnp.float32),
          pltpu.VMEM_SHARED((NPAD,), jnp.float32),
      ],
  )
  def deg_kernel(s0_hbm, d0_hbm, s1_hbm, d1_hbm, o0, o1, o2, o3,
                 idx_v, ones_v, zeros_v, h0, h1, h2, h3):
    core = lax.axis_index("c")
    sub = lax.axis_index("s")
    wid = sub * NC + core
    hists = (h0, h1, h2, h3)
    outs = (o0, o1, o2, o3)

    @pl.loop(0, C // 16)
    def _(t):
      ones_v[pl.ds(t * 16, 16)] = jnp.ones((16,), jnp.float32)

    @pl.loop(0, RPT // 16)
    def _(t):
      zeros_v[pl.ds(t * 16, 16)] = jnp.zeros((16,), jnp.float32)

    for h in hists:
      pltpu.sync_copy(zeros_v, h.at[pl.ds(sub * RPT, RPT)])

    for k, src in enumerate((s0_hbm, d0_hbm, s1_hbm, d1_hbm)):
      pltpu.sync_copy(src.at[pl.ds(wid * NCHT, NCHT)],
                      idx_v.at[pl.ds(k * NCHT, NCHT)])

    plsc.subcore_barrier()

    for k, h in enumerate(hists):
      @pl.loop(0, NCHT)
      def _(j, k=k, h=h):
        pltpu.sync_copy(ones_v, h.at[idx_v.at[k * NCHT + j]], add=True)

    plsc.subcore_barrier()

    for h, o in zip(hists, outs):
      pltpu.sync_copy(h.at[pl.ds(sub * RPT, RPT)],
                      o.at[pl.ds(core * NPAD + sub * RPT, RPT)])

  return deg_kernel(s0, d0, s1, d1)


def _sc_aggregate(h, src, dst):
  @functools.partial(
      pl.kernel,
      out_type=jax.ShapeDtypeStruct((2, NPAD, D), jnp.float32),
      mesh=_mesh,
      scratch_types=[
          pltpu.VMEM((NCH, C), jnp.int32),
          pltpu.VMEM((NCH, C), jnp.int32),
          pltpu.VMEM((2 * C, D), jnp.float32),
          pltpu.VMEM_SHARED((NPAD, D), jnp.float32),
          pltpu.SemaphoreType.DMA,
      ],
  )
  def agg_kernel(h_hbm, src_hbm, dst_hbm, out_hbm,
                 sidx, didx, buf, acc, sem0):
    core = lax.axis_index("c")
    sub = lax.axis_index("s")
    wid = sub * NC + core

    pltpu.sync_copy(src_hbm.at[pl.ds(wid * NCH, NCH)], sidx)
    pltpu.sync_copy(dst_hbm.at[pl.ds(wid * NCH, NCH)], didx)

    @pl.loop(0, C)
    def _(r):
      @pl.loop(0, D // 16)
      def _(t):
        buf.at[pl.ds(r, 1), pl.ds(t * 16, 16)][...] = (
            jnp.zeros((1, 16), jnp.float32))

    @pl.loop(0, RPT // C)
    def _(b):
      pltpu.sync_copy(buf.at[pl.ds(0, C)],
                      acc.at[pl.ds(sub * RPT + b * C, C)])

    plsc.subcore_barrier()

    @pl.loop(0, NCH + 1)
    def _(i):
      @pl.when(i < NCH)
      def _():
        pltpu.async_copy(h_hbm.at[sidx.at[i]],
                         buf.at[pl.ds((i % 2) * C, C)], sem0)

      @pl.when(i >= 1)
      def _():
        j = i - 1
        sl = buf.at[pl.ds((j % 2) * C, C)]
        pltpu.make_async_copy(h_hbm.at[sidx.at[0]], sl, sem0).wait()
        pltpu.sync_copy(sl, acc.at[didx.at[j]], add=True)

    plsc.subcore_barrier()

    pltpu.sync_copy(acc.at[pl.ds(sub * RPT, RPT)],
                    out_hbm.at[core, pl.ds(sub * RPT, RPT)])

  return agg_kernel(h, src, dst)


BM = 1024


def _norm(p_ref):
  deg = p_ref[0] + p_ref[1]
  n = jnp.where(deg > 0, lax.rsqrt(jnp.maximum(deg, 1.0)), 0.0)
  return n.reshape(-1, 1)


_DEG_SPEC = pl.BlockSpec((2, BM), lambda i: (0, i))


def _mm1_body(x_ref, w_ref, ds_ref, o_ref):
  o_ref[...] = jnp.dot(x_ref[...], w_ref[...],
                       preferred_element_type=jnp.float32) * _norm(ds_ref)


def _tc_mm1(x, W, degs):
  return pl.pallas_call(
      _mm1_body,
      grid=(NPAD // BM,),
      in_specs=[
          pl.BlockSpec((BM, D), lambda i: (i, 0)),
          pl.BlockSpec((D, D), lambda i: (0, 0)),
          _DEG_SPEC,
      ],
      out_specs=pl.BlockSpec((BM, D), lambda i: (i, 0)),
      out_shape=jax.ShapeDtypeStruct((NPAD, D), jnp.float32),
  )(x, W, degs)


def _mm2_body(pa_ref, pb_ref, dd_ref, b_ref, w_ref, ds_ref, o_ref):
  agg = pa_ref[0] + pa_ref[1] + pb_ref[0] + pb_ref[1]
  x2 = jnp.maximum(agg * _norm(dd_ref) + b_ref[...], 0.0)
  o_ref[...] = jnp.dot(x2, w_ref[...],
                       preferred_element_type=jnp.float32) * _norm(ds_ref)


def _tc_mm2(pa, pb, degd, b, W, degs):
  return pl.pallas_call(
      _mm2_body,
      grid=(NPAD // BM,),
      in_specs=[
          pl.BlockSpec((2, BM, D), lambda i: (0, i, 0)),
          pl.BlockSpec((2, BM, D), lambda i: (0, i, 0)),
          _DEG_SPEC,
          pl.BlockSpec((1, D), lambda i: (0, 0)),
          pl.BlockSpec((D, D), lambda i: (0, 0)),
          _DEG_SPEC,
      ],
      out_specs=pl.BlockSpec((BM, D), lambda i: (i, 0)),
      out_shape=jax.ShapeDtypeStruct((NPAD, D), jnp.float32),
  )(pa, pb, degd, b, W, degs)


def _fin_body(pa_ref, pb_ref, dd_ref, b_ref, o_ref):
  agg = pa_ref[0] + pa_ref[1] + pb_ref[0] + pb_ref[1]
  o_ref[...] = agg * _norm(dd_ref) + b_ref[...]


def _tc_fin(pa, pb, degd, b):
  return pl.pallas_call(
      _fin_body,
      grid=(NPAD // BM,),
      in_specs=[
          pl.BlockSpec((2, BM, D), lambda i: (0, i, 0)),
          pl.BlockSpec((2, BM, D), lambda i: (0, i, 0)),
          _DEG_SPEC,
          pl.BlockSpec((1, D), lambda i: (0, 0)),
      ],
      out_specs=pl.BlockSpec((BM, D), lambda i: (i, 0)),
      out_shape=jax.ShapeDtypeStruct((NPAD, D), jnp.float32),
  )(pa, pb, degd, b)


def kernel(features, edge_index0, edge_index1, W1, b1, W2, b2):
  x = jnp.pad(features, ((0, NPAD - N), (0, 0)))
  pad_ids = (N + jnp.arange(EPAD - E, dtype=jnp.int32) % (NPAD - N))

  def prep(idx):
    return jnp.concatenate([idx, pad_ids]).reshape(ROWS, C)

  s0 = prep(edge_index0[0])
  d0 = prep(edge_index0[1])
  s1 = prep(edge_index1[0])
  d1 = prep(edge_index1[1])

  degp = _sc_degrees(s0, d0, s1, d1)
  dg = [p.reshape(2, NPAD) for p in degp]

  h1 = _tc_mm1(x, W1, dg[0])
  p1a = _sc_aggregate(h1, s0[:ROWSH], d0[:ROWSH])
  p1b = _sc_aggregate(h1, s0[ROWSH:], d0[ROWSH:])
  h2 = _tc_mm2(p1a, p1b, dg[1], b1.reshape(1, D), W2, dg[2])
  p2a = _sc_aggregate(h2, s1[:ROWSH], d1[:ROWSH])
  p2b = _sc_aggregate(h2, s1[ROWSH:], d1[ROWSH:])
  out = _tc_fin(p2a, p2b, dg[3], b2.reshape(1, D))
  return out[:N]
